# Initial kernel scaffold; baseline (speedup 1.0000x reference)
import jax
import jax.numpy as jnp
from jax.experimental import pallas as pl


def kernel(x, edge_index, W1, b1, W2, b2, W_ih, b_ih, W_hh, b_hh, gamma, beta):
    raise NotImplementedError("write your pallas kernel here")


# speedup vs baseline: 2.9800x; 2.9800x over previous
import functools

import jax
import jax.numpy as jnp
from jax import lax
from jax.experimental import pallas as pl
from jax.experimental.pallas import tpu as pltpu
from jax.experimental.pallas import tpu_sc as plsc

N = 10000
E = 160000
D = 256
H = 128
WACC = 128
CROWS = 80
NACC = N + CROWS
NS = 16
NC = 2
EPT = E // NS
CH = 80
NCHUNK = EPT // CH
RPT = 624
TAIL = N - NS * RPT


def _stageA_body(x_ref, w_ref, b_ref, t_ref):
    t_ref[...] = (
        jnp.dot(x_ref[...], w_ref[0], preferred_element_type=jnp.float32)
        + b_ref[0]
    )


def _stageA(x, wstk, bstk):
    blk = 400
    grid = (4, N // blk)
    return pl.pallas_call(
        _stageA_body,
        grid=grid,
        in_specs=[
            pl.BlockSpec((blk, D), lambda i, j: (j, 0)),
            pl.BlockSpec((1, D, H), lambda i, j: (i, 0, 0)),
            pl.BlockSpec((1, 1, H), lambda i, j: (i, 0, 0)),
        ],
        out_specs=pl.BlockSpec((blk, H), lambda i, j: (i * (N // blk) + j, 0)),
        out_shape=jax.ShapeDtypeStruct((4 * N, H), jnp.float32),
    )(x, wstk, bstk)


def _stageB_tile(t_hbm, idxp_hbm, idxq_hbm, idxs_hbm, acc_hbm,
                 idxp_c, idxq_c, idxs_c, ic_v, p_v, q_v, s_v,
                 shared, sem0, sem1, sem2):
    c = lax.axis_index("c")
    s = lax.axis_index("s")

    zv = jnp.zeros((16,), jnp.float32)

    def zero_row(k, carry):
        for l in range(WACC // 16):
            s_v[k, pl.ds(l * 16, 16)] = zv
        return carry

    lax.fori_loop(0, CH, zero_row, 0)

    for k in range(CROWS // 16):
        ic_v[0, pl.ds(k * 16, 16)] = lax.iota(jnp.int32, 16) + (N + k * 16)

    base = s * RPT

    def zero_acc(k, carry):
        pltpu.sync_copy(s_v, shared.at[pl.ds(base + k * CH, CH)])
        return carry

    lax.fori_loop(0, 7, zero_acc, 0)
    pltpu.sync_copy(s_v.at[pl.ds(0, 64)],
                    shared.at[pl.ds(base + 7 * CH, 64)])

    @pl.when(s == NS - 1)
    def _():
        pltpu.sync_copy(s_v.at[pl.ds(0, TAIL)],
                        shared.at[pl.ds(NS * RPT, TAIL)])

    @pl.when(s == 0)
    def _():
        pltpu.sync_copy(s_v, shared.at[pl.ds(N, CROWS)])

    plsc.subcore_barrier()

    def chunk(j, carry):
        ci = pltpu.async_copy(idxp_hbm.at[c, s, pl.ds(j, 1)], idxp_c, sem0)
        cj = pltpu.async_copy(idxq_hbm.at[c, s, pl.ds(j, 1)], idxq_c, sem1)
        ck = pltpu.async_copy(idxs_hbm.at[s, pl.ds(j, 1)], idxs_c, sem2)
        ci.wait()
        cj.wait()
        cp = pltpu.async_copy(t_hbm.at[idxp_c.at[0]], p_v, sem0)
        cq = pltpu.async_copy(t_hbm.at[idxq_c.at[0]], q_v, sem1)
        cp.wait()
        cq.wait()

        def row(k, rc):
            for l in range(H // 16):
                p = p_v[k, pl.ds(l * 16, 16)]
                q = q_v[k, pl.ds(l * 16, 16)]
                h = p + q
                s_v[k, pl.ds(l * 16, 16)] = h / (1.0 + jnp.exp(-h))
            return rc

        lax.fori_loop(0, CH, row, 0)
        ck.wait()
        pltpu.sync_copy(s_v, shared.at[idxs_c.at[0]], add=True)
        return carry

    lax.fori_loop(0, NCHUNK, chunk, 0)

    lax.fori_loop(0, CH, zero_row, 0)
    ones = jnp.ones((16,), jnp.float32)

    def hist(j, carry):
        pltpu.async_copy(idxs_hbm.at[s, pl.ds(j, 1)], idxs_c, sem2).wait()
        for k in range(CH // 16):
            iv = idxs_c[0, pl.ds(k * 16, 16)]
            plsc.addupdate_scatter(
                s_v, [lax.shift_right_logical(iv, 7), iv & 127], ones)
        return carry

    lax.fori_loop(0, NCHUNK, hist, 0)
    pltpu.sync_copy(s_v, shared.at[ic_v.at[0]], add=True)

    plsc.subcore_barrier()

    out_base = c * NACC + base
    pltpu.sync_copy(shared.at[pl.ds(base, RPT)],
                    acc_hbm.at[pl.ds(out_base, RPT)])

    @pl.when(s == NS - 1)
    def _():
        pltpu.sync_copy(shared.at[pl.ds(NS * RPT, TAIL + CROWS)],
                        acc_hbm.at[pl.ds(c * NACC + NS * RPT, TAIL + CROWS)])


def _stageB(t_tab, idxp, idxq, idxs):
    mesh = plsc.VectorSubcoreMesh(core_axis_name="c", subcore_axis_name="s")
    f = pl.kernel(
        _stageB_tile,
        out_type=jax.ShapeDtypeStruct((NC * NACC, WACC), jnp.float32),
        mesh=mesh,
        compiler_params=pltpu.CompilerParams(needs_layout_passes=False),
        scratch_types=[
            pltpu.VMEM((1, CH), jnp.int32),
            pltpu.VMEM((1, CH), jnp.int32),
            pltpu.VMEM((1, CH), jnp.int32),
            pltpu.VMEM((1, CROWS), jnp.int32),
            pltpu.VMEM((CH, H), jnp.float32),
            pltpu.VMEM((CH, H), jnp.float32),
            pltpu.VMEM((CH, WACC), jnp.float32),
            pltpu.VMEM_SHARED((NACC, WACC), jnp.float32),
            pltpu.SemaphoreType.DMA,
            pltpu.SemaphoreType.DMA,
            pltpu.SemaphoreType.DMA,
        ],
    )
    return f(t_tab, idxp, idxq, idxs)


def _stageC_body(s_ref, cnt_ref, x_ref, w2_ref, b2_ref, wih_ref, bih_ref,
                 whh_ref, bhh_ref, g_ref, be_ref, o_ref):
    cnt = cnt_ref[...]
    agg_sum = (
        jnp.dot(s_ref[...], w2_ref[...], preferred_element_type=jnp.float32)
        + cnt * b2_ref[...]
    )
    agg = agg_sum / (cnt + 1e-8)
    xb = x_ref[...]
    gi = jnp.dot(agg, wih_ref[...], preferred_element_type=jnp.float32) + bih_ref[...]
    gh = jnp.dot(xb, whh_ref[...], preferred_element_type=jnp.float32) + bhh_ref[...]
    r = jax.nn.sigmoid(gi[:, :D] + gh[:, :D])
    z = jax.nn.sigmoid(gi[:, D:2 * D] + gh[:, D:2 * D])
    nn_ = jnp.tanh(gi[:, 2 * D:] + r * gh[:, 2 * D:])
    xn = (1.0 - z) * nn_ + z * xb
    mean = jnp.mean(xn, axis=-1, keepdims=True)
    xc = xn - mean
    var = jnp.mean(xc * xc, axis=-1, keepdims=True)
    o_ref[...] = xc * jax.lax.rsqrt(var + 1e-5) * g_ref[...] + be_ref[...]


def _stageC(s_mat, cnt, x, w2t, b2, wiht, bih, whht, bhh, gamma, beta):
    blk = 400
    grid = (N // blk,)
    full = lambda shape: pl.BlockSpec(shape, lambda i: tuple(0 for _ in shape))
    return pl.pallas_call(
        _stageC_body,
        grid=grid,
        in_specs=[
            pl.BlockSpec((blk, D), lambda i: (i, 0)),
            pl.BlockSpec((blk, 1), lambda i: (i, 0)),
            pl.BlockSpec((blk, D), lambda i: (i, 0)),
            full((D, D)),
            full((1, D)),
            full((D, 3 * D)),
            full((1, 3 * D)),
            full((D, 3 * D)),
            full((1, 3 * D)),
            full((1, D)),
            full((1, D)),
        ],
        out_specs=pl.BlockSpec((blk, D), lambda i: (i, 0)),
        out_shape=jax.ShapeDtypeStruct((N, D), jnp.float32),
    )(s_mat, cnt, x, w2t, b2, wiht, bih, whht, bhh, gamma, beta)


def kernel(x, edge_index, W1, b1, W2, b2, W_ih, b_ih, W_hh, b_hh, gamma, beta):
    row = edge_index[0].astype(jnp.int32)
    col = edge_index[1].astype(jnp.int32)

    w1t = W1.T
    wstk = jnp.stack([
        w1t[:D, :H], w1t[:D, H:], w1t[D:, :H], w1t[D:, H:],
    ])
    zh = jnp.zeros((H,), jnp.float32)
    bstk = jnp.stack([b1[:H], b1[H:], zh, zh]).reshape(4, 1, H)
    t_tab = _stageA(x, wstk, bstk)

    er = row.reshape(NS, NCHUNK, CH)
    ec = col.reshape(NS, NCHUNK, CH)
    idxp = jnp.stack([er, er + N])
    idxq = jnp.stack([ec + 2 * N, ec + 3 * N])

    acc = _stageB(t_tab, idxp, idxq, er)

    s_mat = jnp.concatenate([acc[:N], acc[NACC:NACC + N]], axis=1)
    cnt = acc[N:NACC].reshape(-1)[:N].reshape(N, 1)

    return _stageC(
        s_mat, cnt, x,
        W2.T, b2.reshape(1, D),
        W_ih.T, b_ih.reshape(1, 3 * D),
        W_hh.T, b_hh.reshape(1, 3 * D),
        gamma.reshape(1, D), beta.reshape(1, D),
    )


# speedup vs baseline: 3.5947x; 1.2063x over previous
import functools

import jax
import jax.numpy as jnp
from jax import lax
from jax.experimental import pallas as pl
from jax.experimental.pallas import tpu as pltpu
from jax.experimental.pallas import tpu_sc as plsc

N = 10000
E = 160000
D = 256
H = 128
WACC = 128
CROWS = 80
NACC = N + CROWS
NS = 16
NC = 2
EPT = E // NS
CH = 80
NCHUNK = EPT // CH
RPT = 624
TAIL = N - NS * RPT


def _stageA_body(x_ref, w_ref, b_ref, t_ref):
    t_ref[...] = (
        jnp.dot(x_ref[...], w_ref[0], preferred_element_type=jnp.float32)
        + b_ref[0]
    )


def _stageA(x, wstk, bstk):
    blk = 400
    grid = (4, N // blk)
    return pl.pallas_call(
        _stageA_body,
        grid=grid,
        in_specs=[
            pl.BlockSpec((blk, D), lambda i, j: (j, 0)),
            pl.BlockSpec((1, D, H), lambda i, j: (i, 0, 0)),
            pl.BlockSpec((1, 1, H), lambda i, j: (i, 0, 0)),
        ],
        out_specs=pl.BlockSpec((blk, H), lambda i, j: (i * (N // blk) + j, 0)),
        out_shape=jax.ShapeDtypeStruct((4 * N, H), jnp.float32),
    )(x, wstk, bstk)


def _stageB_tile(t_hbm, idxpack_hbm, idxs_hbm, acc_hbm,
                 ix0, ix1, ic_v, p0, p1, q_v, h_v,
                 shared, semp0, semp1, semq, semsc0, semsc1, semix, semh):
    c = lax.axis_index("c")
    s = lax.axis_index("s")
    ixs = (ix0, ix1)
    ps = (p0, p1)
    semps = (semp0, semp1)
    semscs = (semsc0, semsc1)

    zv = jnp.zeros((16,), jnp.float32)

    def zero_p0(k, carry):
        for l in range(WACC // 16):
            p0[k, pl.ds(l * 16, 16)] = zv
        return carry

    lax.fori_loop(0, CH, zero_p0, 0)

    for k in range(CROWS // 16):
        ic_v[0, pl.ds(k * 16, 16)] = lax.iota(jnp.int32, 16) + (N + k * 16)

    base = s * RPT

    def zero_acc(k, carry):
        pltpu.sync_copy(p0, shared.at[pl.ds(base + k * CH, CH)])
        return carry

    lax.fori_loop(0, 7, zero_acc, 0)
    pltpu.sync_copy(p0.at[pl.ds(0, 64)],
                    shared.at[pl.ds(base + 7 * CH, 64)])

    @pl.when(s == NS - 1)
    def _():
        pltpu.sync_copy(p0.at[pl.ds(0, TAIL)],
                        shared.at[pl.ds(NS * RPT, TAIL)])

    @pl.when(s == 0)
    def _():
        pltpu.sync_copy(p0, shared.at[pl.ds(N, CROWS)])

    plsc.subcore_barrier()

    def compute_inplace(pb, qb):
        def row(k, rc):
            for l in range(H // 16):
                p = pb[k, pl.ds(l * 16, 16)]
                q = qb[k, pl.ds(l * 16, 16)]
                h = p + q
                pb[k, pl.ds(l * 16, 16)] = h / (1.0 + jnp.exp(-h))
            return rc

        lax.fori_loop(0, CH, row, 0)

    def wait_scatter(b):
        pltpu.make_async_copy(ps[b], shared.at[pl.ds(0, CH)],
                              semscs[b]).wait()

    pltpu.sync_copy(idxpack_hbm.at[c, s, 0], ix0)
    pltpu.async_copy(t_hbm.at[ix0.at[0]], p0, semp0)
    pltpu.async_copy(t_hbm.at[ix0.at[1]], q_v, semq)

    def two_chunks(j2, carry):
        for b in (0, 1):
            j = 2 * j2 + b
            nb = 1 - b
            if b == 0:
                @pl.when(j2 > 0)
                def _():
                    wait_scatter(nb)
            else:
                wait_scatter(nb)
            pltpu.sync_copy(idxpack_hbm.at[c, s, j + 1], ixs[nb])
            pltpu.async_copy(t_hbm.at[ixs[nb].at[0]], ps[nb], semps[nb])
            pltpu.make_async_copy(t_hbm.at[pl.ds(0, CH)], ps[b],
                                  semps[b]).wait()
            pltpu.make_async_copy(t_hbm.at[pl.ds(0, CH)], q_v, semq).wait()
            compute_inplace(ps[b], q_v)
            pltpu.async_copy(t_hbm.at[ixs[nb].at[1]], q_v, semq)
            pltpu.async_copy(ps[b], shared.at[ixs[b].at[2]], semscs[b],
                             add=True)
        return carry

    lax.fori_loop(0, (NCHUNK - 1) // 2, two_chunks, 0)

    pltpu.make_async_copy(t_hbm.at[pl.ds(0, CH)], p0, semp0).wait()
    pltpu.make_async_copy(t_hbm.at[pl.ds(0, CH)], q_v, semq).wait()
    compute_inplace(p0, q_v)
    wait_scatter(1)
    pltpu.sync_copy(p0, shared.at[ix0.at[2]], add=True)

    def zero_q(k, carry):
        for l in range(WACC // 16):
            q_v[k, pl.ds(l * 16, 16)] = zv
        return carry

    lax.fori_loop(0, CH, zero_q, 0)
    ones = jnp.ones((16,), jnp.float32)

    def hist(g, carry):
        pltpu.async_copy(idxs_hbm.at[s, g], h_v, semh).wait()
        def hrow(r, rc):
            for k in range(CH // 16):
                iv = h_v[r, pl.ds(k * 16, 16)]
                plsc.addupdate_scatter(
                    q_v, [lax.shift_right_logical(iv, 7), iv & 127], ones)
            return rc
        lax.fori_loop(0, 5, hrow, 0)
        return carry

    lax.fori_loop(0, NCHUNK // 5, hist, 0)
    pltpu.sync_copy(q_v, shared.at[ic_v.at[0]], add=True)

    plsc.subcore_barrier()

    out_base = c * NACC + base
    pltpu.sync_copy(shared.at[pl.ds(base, RPT)],
                    acc_hbm.at[pl.ds(out_base, RPT)])

    @pl.when(s == NS - 1)
    def _():
        pltpu.sync_copy(shared.at[pl.ds(NS * RPT, TAIL + CROWS)],
                        acc_hbm.at[pl.ds(c * NACC + NS * RPT, TAIL + CROWS)])


def _stageB(t_tab, idxpack, idxs):
    mesh = plsc.VectorSubcoreMesh(core_axis_name="c", subcore_axis_name="s")
    f = pl.kernel(
        _stageB_tile,
        out_type=jax.ShapeDtypeStruct((NC * NACC, WACC), jnp.float32),
        mesh=mesh,
        compiler_params=pltpu.CompilerParams(needs_layout_passes=False),
        scratch_types=[
            pltpu.VMEM((3, CH), jnp.int32),
            pltpu.VMEM((3, CH), jnp.int32),
            pltpu.VMEM((1, CROWS), jnp.int32),
            pltpu.VMEM((CH, H), jnp.float32),
            pltpu.VMEM((CH, H), jnp.float32),
            pltpu.VMEM((CH, WACC), jnp.float32),
            pltpu.VMEM((5, CH), jnp.int32),
            pltpu.VMEM_SHARED((NACC, WACC), jnp.float32),
            pltpu.SemaphoreType.DMA,
            pltpu.SemaphoreType.DMA,
            pltpu.SemaphoreType.DMA,
            pltpu.SemaphoreType.DMA,
            pltpu.SemaphoreType.DMA,
            pltpu.SemaphoreType.DMA,
            pltpu.SemaphoreType.DMA,
        ],
    )
    return f(t_tab, idxpack, idxs)


def _stageC_body(s_ref, cnt_ref, x_ref, w2_ref, b2_ref, wih_ref, bih_ref,
                 whh_ref, bhh_ref, g_ref, be_ref, o_ref):
    cnt = cnt_ref[...]
    agg_sum = (
        jnp.dot(s_ref[...], w2_ref[...], preferred_element_type=jnp.float32)
        + cnt * b2_ref[...]
    )
    agg = agg_sum / (cnt + 1e-8)
    xb = x_ref[...]
    gi = jnp.dot(agg, wih_ref[...], preferred_element_type=jnp.float32) + bih_ref[...]
    gh = jnp.dot(xb, whh_ref[...], preferred_element_type=jnp.float32) + bhh_ref[...]
    r = jax.nn.sigmoid(gi[:, :D] + gh[:, :D])
    z = jax.nn.sigmoid(gi[:, D:2 * D] + gh[:, D:2 * D])
    nn_ = jnp.tanh(gi[:, 2 * D:] + r * gh[:, 2 * D:])
    xn = (1.0 - z) * nn_ + z * xb
    mean = jnp.mean(xn, axis=-1, keepdims=True)
    xc = xn - mean
    var = jnp.mean(xc * xc, axis=-1, keepdims=True)
    o_ref[...] = xc * jax.lax.rsqrt(var + 1e-5) * g_ref[...] + be_ref[...]


def _stageC(s_mat, cnt, x, w2t, b2, wiht, bih, whht, bhh, gamma, beta):
    blk = 400
    grid = (N // blk,)
    full = lambda shape: pl.BlockSpec(shape, lambda i: tuple(0 for _ in shape))
    return pl.pallas_call(
        _stageC_body,
        grid=grid,
        in_specs=[
            pl.BlockSpec((blk, D), lambda i: (i, 0)),
            pl.BlockSpec((blk, 1), lambda i: (i, 0)),
            pl.BlockSpec((blk, D), lambda i: (i, 0)),
            full((D, D)),
            full((1, D)),
            full((D, 3 * D)),
            full((1, 3 * D)),
            full((D, 3 * D)),
            full((1, 3 * D)),
            full((1, D)),
            full((1, D)),
        ],
        out_specs=pl.BlockSpec((blk, D), lambda i: (i, 0)),
        out_shape=jax.ShapeDtypeStruct((N, D), jnp.float32),
    )(s_mat, cnt, x, w2t, b2, wiht, bih, whht, bhh, gamma, beta)


def kernel(x, edge_index, W1, b1, W2, b2, W_ih, b_ih, W_hh, b_hh, gamma, beta):
    row = edge_index[0].astype(jnp.int32)
    col = edge_index[1].astype(jnp.int32)

    w1t = W1.T
    wstk = jnp.stack([
        w1t[:D, :H], w1t[:D, H:], w1t[D:, :H], w1t[D:, H:],
    ])
    zh = jnp.zeros((H,), jnp.float32)
    bstk = jnp.stack([b1[:H], b1[H:], zh, zh]).reshape(4, 1, H)
    t_tab = _stageA(x, wstk, bstk)

    er = row.reshape(NS, NCHUNK, CH)
    ec = col.reshape(NS, NCHUNK, CH)
    pack0 = jnp.stack([er, ec + 2 * N, er], axis=2)
    pack1 = jnp.stack([er + N, ec + 3 * N, er], axis=2)
    idxpack = jnp.stack([pack0, pack1])

    acc = _stageB(t_tab, idxpack,
                  er.reshape(NS, NCHUNK // 5, 5, CH))

    s_mat = jnp.concatenate([acc[:N], acc[NACC:NACC + N]], axis=1)
    cnt = acc[N:NACC].reshape(-1)[:N].reshape(N, 1)

    return _stageC(
        s_mat, cnt, x,
        W2.T, b2.reshape(1, D),
        W_ih.T, b_ih.reshape(1, 3 * D),
        W_hh.T, b_hh.reshape(1, 3 * D),
        gamma.reshape(1, D), beta.reshape(1, D),
    )


# speedup vs baseline: 3.6244x; 1.0083x over previous
import functools

import jax
import jax.numpy as jnp
from jax import lax
from jax.experimental import pallas as pl
from jax.experimental.pallas import tpu as pltpu
from jax.experimental.pallas import tpu_sc as plsc

N = 10000
E = 160000
D = 256
H = 128
WACC = 128
CROWS = 80
NACC = N + CROWS
NS = 16
NC = 2
EPT = E // NS
CH = 80
NCHUNK = EPT // CH
RPT = 624
TAIL = N - NS * RPT


def _stageA_body(x_ref, w_ref, b_ref, t_ref):
    t_ref[...] = (
        jnp.dot(x_ref[...], w_ref[0], preferred_element_type=jnp.float32)
        + b_ref[0]
    )


def _stageA(x, wstk, bstk):
    blk = 400
    grid = (4, N // blk)
    return pl.pallas_call(
        _stageA_body,
        grid=grid,
        in_specs=[
            pl.BlockSpec((blk, D), lambda i, j: (j, 0)),
            pl.BlockSpec((1, D, H), lambda i, j: (i, 0, 0)),
            pl.BlockSpec((1, 1, H), lambda i, j: (i, 0, 0)),
        ],
        out_specs=pl.BlockSpec((blk, H), lambda i, j: (i * (N // blk) + j, 0)),
        out_shape=jax.ShapeDtypeStruct((4 * N, H), jnp.float32),
    )(x, wstk, bstk)


def _stageB_tile(t_hbm, idxpack_hbm, idxs_hbm, feat_hbm, cnt_hbm,
                 ix0, ix1, ic_v, p0, p1, q_v, h_v,
                 shared, semp0, semp1, semq, semsc0, semsc1, semix, semh):
    c = lax.axis_index("c")
    s = lax.axis_index("s")
    ixs = (ix0, ix1)
    ps = (p0, p1)
    semps = (semp0, semp1)
    semscs = (semsc0, semsc1)

    zv = jnp.zeros((16,), jnp.float32)

    def zero_p0(k, carry):
        for l in range(WACC // 16):
            p0[k, pl.ds(l * 16, 16)] = zv
        return carry

    lax.fori_loop(0, CH, zero_p0, 0)

    for k in range(CROWS // 16):
        ic_v[0, pl.ds(k * 16, 16)] = lax.iota(jnp.int32, 16) + (N + k * 16)

    base = s * RPT

    def zero_acc(k, carry):
        pltpu.sync_copy(p0, shared.at[pl.ds(base + k * CH, CH)])
        return carry

    lax.fori_loop(0, 7, zero_acc, 0)
    pltpu.sync_copy(p0.at[pl.ds(0, 64)],
                    shared.at[pl.ds(base + 7 * CH, 64)])

    @pl.when(s == NS - 1)
    def _():
        pltpu.sync_copy(p0.at[pl.ds(0, TAIL)],
                        shared.at[pl.ds(NS * RPT, TAIL)])

    @pl.when(s == 0)
    def _():
        pltpu.sync_copy(p0, shared.at[pl.ds(N, CROWS)])

    plsc.subcore_barrier()

    def compute_inplace(pb, qb):
        def row(k, rc):
            for l in range(H // 16):
                p = pb[k, pl.ds(l * 16, 16)]
                q = qb[k, pl.ds(l * 16, 16)]
                h = p + q
                pb[k, pl.ds(l * 16, 16)] = h / (1.0 + jnp.exp(-h))
            return rc

        lax.fori_loop(0, CH, row, 0)

    def wait_scatter(b):
        pltpu.make_async_copy(ps[b], shared.at[pl.ds(0, CH)],
                              semscs[b]).wait()

    pltpu.sync_copy(idxpack_hbm.at[c, s, 0], ix0)
    pltpu.async_copy(t_hbm.at[ix0.at[0]], p0, semp0)
    pltpu.async_copy(t_hbm.at[ix0.at[1]], q_v, semq)

    def two_chunks(j2, carry):
        for b in (0, 1):
            j = 2 * j2 + b
            nb = 1 - b
            if b == 0:
                @pl.when(j2 > 0)
                def _():
                    wait_scatter(nb)
            else:
                wait_scatter(nb)
            pltpu.sync_copy(idxpack_hbm.at[c, s, j + 1], ixs[nb])
            pltpu.async_copy(t_hbm.at[ixs[nb].at[0]], ps[nb], semps[nb])
            pltpu.make_async_copy(t_hbm.at[pl.ds(0, CH)], ps[b],
                                  semps[b]).wait()
            pltpu.make_async_copy(t_hbm.at[pl.ds(0, CH)], q_v, semq).wait()
            compute_inplace(ps[b], q_v)
            pltpu.async_copy(t_hbm.at[ixs[nb].at[1]], q_v, semq)
            pltpu.async_copy(ps[b], shared.at[ixs[b].at[2]], semscs[b],
                             add=True)
        return carry

    lax.fori_loop(0, (NCHUNK - 1) // 2, two_chunks, 0)

    pltpu.make_async_copy(t_hbm.at[pl.ds(0, CH)], p0, semp0).wait()
    pltpu.make_async_copy(t_hbm.at[pl.ds(0, CH)], q_v, semq).wait()
    compute_inplace(p0, q_v)
    wait_scatter(1)
    pltpu.sync_copy(p0, shared.at[ix0.at[2]], add=True)

    def zero_q(k, carry):
        for l in range(WACC // 16):
            q_v[k, pl.ds(l * 16, 16)] = zv
        return carry

    lax.fori_loop(0, CH, zero_q, 0)
    ones = jnp.ones((16,), jnp.float32)

    def hist(g, carry):
        pltpu.async_copy(idxs_hbm.at[s, g], h_v, semh).wait()
        def hrow(r, rc):
            for k in range(CH // 16):
                iv = h_v[r, pl.ds(k * 16, 16)]
                plsc.addupdate_scatter(
                    q_v, [lax.shift_right_logical(iv, 7), iv & 127], ones)
            return rc
        lax.fori_loop(0, 5, hrow, 0)
        return carry

    lax.fori_loop(0, NCHUNK // 5, hist, 0)
    pltpu.sync_copy(q_v, shared.at[ic_v.at[0]], add=True)

    plsc.subcore_barrier()

    pltpu.sync_copy(shared.at[pl.ds(base, RPT)],
                    feat_hbm.at[c, pl.ds(base, RPT)])

    @pl.when(s == NS - 1)
    def _():
        pltpu.sync_copy(shared.at[pl.ds(NS * RPT, TAIL)],
                        feat_hbm.at[c, pl.ds(NS * RPT, TAIL)])
        pltpu.sync_copy(shared.at[pl.ds(N, CROWS)], cnt_hbm.at[c])


def _stageB(t_tab, idxpack, idxs):
    mesh = plsc.VectorSubcoreMesh(core_axis_name="c", subcore_axis_name="s")
    f = pl.kernel(
        _stageB_tile,
        out_type=(jax.ShapeDtypeStruct((NC, N, WACC), jnp.float32),
                  jax.ShapeDtypeStruct((NC, CROWS, WACC), jnp.float32)),
        mesh=mesh,
        compiler_params=pltpu.CompilerParams(needs_layout_passes=False),
        scratch_types=[
            pltpu.VMEM((3, CH), jnp.int32),
            pltpu.VMEM((3, CH), jnp.int32),
            pltpu.VMEM((1, CROWS), jnp.int32),
            pltpu.VMEM((CH, H), jnp.float32),
            pltpu.VMEM((CH, H), jnp.float32),
            pltpu.VMEM((CH, WACC), jnp.float32),
            pltpu.VMEM((5, CH), jnp.int32),
            pltpu.VMEM_SHARED((NACC, WACC), jnp.float32),
            pltpu.SemaphoreType.DMA,
            pltpu.SemaphoreType.DMA,
            pltpu.SemaphoreType.DMA,
            pltpu.SemaphoreType.DMA,
            pltpu.SemaphoreType.DMA,
            pltpu.SemaphoreType.DMA,
            pltpu.SemaphoreType.DMA,
        ],
    )
    return f(t_tab, idxpack, idxs)


def _stageC_body(slo_ref, shi_ref, cnt_ref, x_ref, w2a_ref, w2b_ref, b2_ref,
                 wih_ref, bih_ref, whh_ref, bhh_ref, g_ref, be_ref, o_ref):
    bf = jnp.bfloat16
    cnt = cnt_ref[...]
    agg_sum = (
        jnp.dot(slo_ref[0].astype(bf), w2a_ref[...],
                preferred_element_type=jnp.float32)
        + jnp.dot(shi_ref[0].astype(bf), w2b_ref[...],
                  preferred_element_type=jnp.float32)
        + cnt * b2_ref[...]
    )
    agg = agg_sum / (cnt + 1e-8)
    xb = x_ref[...]
    gi = jnp.dot(agg.astype(bf), wih_ref[...],
                 preferred_element_type=jnp.float32) + bih_ref[...]
    gh = jnp.dot(xb.astype(bf), whh_ref[...],
                 preferred_element_type=jnp.float32) + bhh_ref[...]
    r = jax.nn.sigmoid(gi[:, :D] + gh[:, :D])
    z = jax.nn.sigmoid(gi[:, D:2 * D] + gh[:, D:2 * D])
    nn_ = jnp.tanh(gi[:, 2 * D:] + r * gh[:, 2 * D:])
    xn = (1.0 - z) * nn_ + z * xb
    mean = jnp.mean(xn, axis=-1, keepdims=True)
    xc = xn - mean
    var = jnp.mean(xc * xc, axis=-1, keepdims=True)
    o_ref[...] = xc * jax.lax.rsqrt(var + 1e-5) * g_ref[...] + be_ref[...]


def _stageC(feat, cnt, x, w2a, w2b, b2, wiht, bih, whht, bhh, gamma, beta):
    blk = 400
    grid = (N // blk,)
    full = lambda shape: pl.BlockSpec(shape, lambda i: tuple(0 for _ in shape))
    return pl.pallas_call(
        _stageC_body,
        grid=grid,
        in_specs=[
            pl.BlockSpec((1, blk, H), lambda i: (0, i, 0)),
            pl.BlockSpec((1, blk, H), lambda i: (1, i, 0)),
            pl.BlockSpec((blk, 1), lambda i: (i, 0)),
            pl.BlockSpec((blk, D), lambda i: (i, 0)),
            full((H, D)),
            full((H, D)),
            full((1, D)),
            full((D, 3 * D)),
            full((1, 3 * D)),
            full((D, 3 * D)),
            full((1, 3 * D)),
            full((1, D)),
            full((1, D)),
        ],
        out_specs=pl.BlockSpec((blk, D), lambda i: (i, 0)),
        out_shape=jax.ShapeDtypeStruct((N, D), jnp.float32),
    )(feat, feat, cnt, x, w2a, w2b, b2, wiht, bih, whht, bhh, gamma, beta)


def kernel(x, edge_index, W1, b1, W2, b2, W_ih, b_ih, W_hh, b_hh, gamma, beta):
    row = edge_index[0].astype(jnp.int32)
    col = edge_index[1].astype(jnp.int32)

    w1t = W1.T
    wstk = jnp.stack([
        w1t[:D, :H], w1t[:D, H:], w1t[D:, :H], w1t[D:, H:],
    ])
    zh = jnp.zeros((H,), jnp.float32)
    bstk = jnp.stack([b1[:H], b1[H:], zh, zh]).reshape(4, 1, H)
    t_tab = _stageA(x.astype(jnp.bfloat16), wstk.astype(jnp.bfloat16),
                    bstk)

    er = row.reshape(NS, NCHUNK, CH)
    ec = col.reshape(NS, NCHUNK, CH)
    pack0 = jnp.stack([er, ec + 2 * N, er], axis=2)
    pack1 = jnp.stack([er + N, ec + 3 * N, er], axis=2)
    idxpack = jnp.stack([pack0, pack1])

    feat, cntp = _stageB(t_tab, idxpack,
                         er.reshape(NS, NCHUNK // 5, 5, CH))
    cnt = cntp[0].reshape(-1)[:N].reshape(N, 1)

    bf = jnp.bfloat16
    w2t = W2.T
    return _stageC(
        feat, cnt, x,
        w2t[:H].astype(bf), w2t[H:].astype(bf), b2.reshape(1, D),
        W_ih.T.astype(bf), b_ih.reshape(1, 3 * D),
        W_hh.T.astype(bf), b_hh.reshape(1, 3 * D),
        gamma.reshape(1, D), beta.reshape(1, D),
    )


# speedup vs baseline: 4.2219x; 1.1648x over previous
import functools

import jax
import jax.numpy as jnp
from jax import lax
from jax.experimental import pallas as pl
from jax.experimental.pallas import tpu as pltpu
from jax.experimental.pallas import tpu_sc as plsc

N = 10000
E = 160000
D = 256
H = 128
WACC = 128
CROWS = 80
NACC = N + CROWS
NS = 16
NC = 2
EPT = E // NS
CH = 100
NCHUNK = EPT // CH
IOP = 128
IOQ = 256
IXW = 384
RPT = 624
TAIL = N - NS * RPT
ZB = 80
HCH = 80


def _stageA_body(x_ref, w_ref, b_ref, t_ref):
    t_ref[...] = (
        jnp.dot(x_ref[...], w_ref[0], preferred_element_type=jnp.float32)
        + b_ref[0]
    )


def _stageA(x, wstk, bstk):
    blk = 400
    grid = (4, N // blk)
    return pl.pallas_call(
        _stageA_body,
        grid=grid,
        in_specs=[
            pl.BlockSpec((blk, D), lambda i, j: (j, 0)),
            pl.BlockSpec((1, D, H), lambda i, j: (i, 0, 0)),
            pl.BlockSpec((1, 1, H), lambda i, j: (i, 0, 0)),
        ],
        out_specs=pl.BlockSpec((blk, H), lambda i, j: (i * (N // blk) + j, 0)),
        out_shape=jax.ShapeDtypeStruct((4 * N, H), jnp.float32),
    )(x, wstk, bstk)


def _stageB_tile(t_hbm, idxpack_hbm, idxs_hbm, dummy_hbm, feat_hbm, cnt_hbm,
                 ix0, ix1, ic_v, p0, p1, q_v, h_v,
                 shared, semp0, semp1, semq, semsc0, semsc1, semix, semh):
    c = lax.axis_index("c")
    s = lax.axis_index("s")
    ixs = (ix0, ix1)
    ps = (p0, p1)
    semps = (semp0, semp1)
    semscs = (semsc0, semsc1)

    zv = jnp.zeros((16,), jnp.float32)

    def zero_p0(k, carry):
        for l in range(WACC // 16):
            p0[k, pl.ds(l * 16, 16)] = zv
        return carry

    lax.fori_loop(0, CH, zero_p0, 0)

    for k in range(CROWS // 16):
        ic_v[0, pl.ds(k * 16, 16)] = lax.iota(jnp.int32, 16) + (N + k * 16)

    base = s * RPT

    def zero_acc(k, carry):
        pltpu.sync_copy(p0.at[pl.ds(0, ZB)],
                        shared.at[pl.ds(base + k * ZB, ZB)])
        return carry

    lax.fori_loop(0, 7, zero_acc, 0)
    pltpu.sync_copy(p0.at[pl.ds(0, 64)],
                    shared.at[pl.ds(base + 7 * ZB, 64)])

    @pl.when(s == NS - 1)
    def _():
        pltpu.sync_copy(p0.at[pl.ds(0, TAIL)],
                        shared.at[pl.ds(NS * RPT, TAIL)])

    @pl.when(s == 0)
    def _():
        pltpu.sync_copy(p0.at[pl.ds(0, CROWS)], shared.at[pl.ds(N, CROWS)])

    plsc.subcore_barrier()

    def compute_inplace(pb, qb):
        def row2(k2, rc):
            for r in range(2):
                k = 2 * k2 + r
                for l in range(H // 16):
                    p = pb[k, pl.ds(l * 16, 16)]
                    q = qb[k, pl.ds(l * 16, 16)]
                    h = p + q
                    pb[k, pl.ds(l * 16, 16)] = h / (1.0 + jnp.exp(-h))
            return rc

        lax.fori_loop(0, CH // 2, row2, 0)

    def wait_scatter(b):
        pltpu.make_async_copy(dummy_hbm, ps[b], semscs[b]).wait()

    def ix_sct(b):
        return ixs[b].at[0, pl.ds(0, CH)]

    def ix_p(b):
        return ixs[b].at[0, pl.ds(IOP, CH)]

    def ix_q(b):
        return ixs[b].at[0, pl.ds(IOQ, CH)]

    pltpu.sync_copy(idxpack_hbm.at[c, s, 0], ix0)
    pltpu.async_copy(t_hbm.at[ix_p(0)], p0, semp0)
    pltpu.async_copy(t_hbm.at[ix_q(0)], q_v, semq)

    def two_chunks(j2, carry):
        for b in (0, 1):
            j = 2 * j2 + b
            nb = 1 - b
            if b == 0:
                @pl.when(j2 > 0)
                def _():
                    wait_scatter(nb)
            else:
                wait_scatter(nb)
            pltpu.sync_copy(idxpack_hbm.at[c, s, j + 1], ixs[nb])
            pltpu.async_copy(t_hbm.at[ix_p(nb)], ps[nb], semps[nb])
            pltpu.make_async_copy(dummy_hbm, ps[b], semps[b]).wait()
            pltpu.make_async_copy(dummy_hbm, q_v, semq).wait()
            compute_inplace(ps[b], q_v)
            pltpu.async_copy(t_hbm.at[ix_q(nb)], q_v, semq)
            pltpu.async_copy(ps[b], shared.at[ix_sct(b)], semscs[b],
                             add=True)
        return carry

    lax.fori_loop(0, (NCHUNK - 2) // 2, two_chunks, 0)

    pltpu.make_async_copy(dummy_hbm, p0, semp0).wait()
    pltpu.make_async_copy(dummy_hbm, q_v, semq).wait()
    compute_inplace(p0, q_v)
    wait_scatter(1)
    pltpu.sync_copy(idxpack_hbm.at[c, s, NCHUNK - 1], ix1)
    pltpu.async_copy(t_hbm.at[ix_p(1)], p1, semp1)
    pltpu.async_copy(t_hbm.at[ix_q(1)], q_v, semq)
    pltpu.async_copy(p0, shared.at[ix_sct(0)], semsc0, add=True)
    pltpu.make_async_copy(dummy_hbm, p1, semp1).wait()
    pltpu.make_async_copy(dummy_hbm, q_v, semq).wait()
    compute_inplace(p1, q_v)
    pltpu.sync_copy(p1, shared.at[ix_sct(1)], add=True)
    wait_scatter(0)

    def zero_q(k, carry):
        for l in range(WACC // 16):
            q_v[k, pl.ds(l * 16, 16)] = zv
        return carry

    lax.fori_loop(0, CROWS, zero_q, 0)
    ones = jnp.ones((16,), jnp.float32)

    def hist(g, carry):
        pltpu.async_copy(idxs_hbm.at[s, g], h_v, semh).wait()
        def hrow(r, rc):
            for k in range(HCH // 16):
                iv = h_v[r, pl.ds(k * 16, 16)]
                plsc.addupdate_scatter(
                    q_v, [lax.shift_right_logical(iv, 7), iv & 127], ones)
            return rc
        lax.fori_loop(0, 5, hrow, 0)
        return carry

    lax.fori_loop(0, EPT // (5 * HCH), hist, 0)
    pltpu.sync_copy(q_v.at[pl.ds(0, CROWS)], shared.at[ic_v.at[0]],
                    add=True)

    plsc.subcore_barrier()

    pltpu.sync_copy(shared.at[pl.ds(base, RPT)],
                    feat_hbm.at[c, pl.ds(base, RPT)])

    @pl.when(s == NS - 1)
    def _():
        pltpu.sync_copy(shared.at[pl.ds(NS * RPT, TAIL)],
                        feat_hbm.at[c, pl.ds(NS * RPT, TAIL)])
        pltpu.sync_copy(shared.at[pl.ds(N, CROWS)], cnt_hbm.at[c])


def _stageB(t_tab, idxpack, idxs):
    mesh = plsc.VectorSubcoreMesh(core_axis_name="c", subcore_axis_name="s")
    f = pl.kernel(
        _stageB_tile,
        out_type=(jax.ShapeDtypeStruct((NC, N, WACC), jnp.float32),
                  jax.ShapeDtypeStruct((NC, CROWS, WACC), jnp.float32)),
        mesh=mesh,
        compiler_params=pltpu.CompilerParams(needs_layout_passes=False),
        scratch_types=[
            pltpu.VMEM((1, IXW), jnp.int32),
            pltpu.VMEM((1, IXW), jnp.int32),
            pltpu.VMEM((1, CROWS), jnp.int32),
            pltpu.VMEM((CH, H), jnp.float32),
            pltpu.VMEM((CH, H), jnp.float32),
            pltpu.VMEM((CH, WACC), jnp.float32),
            pltpu.VMEM((5, HCH), jnp.int32),
            pltpu.VMEM_SHARED((NACC, WACC), jnp.float32),
            pltpu.SemaphoreType.DMA,
            pltpu.SemaphoreType.DMA,
            pltpu.SemaphoreType.DMA,
            pltpu.SemaphoreType.DMA,
            pltpu.SemaphoreType.DMA,
            pltpu.SemaphoreType.DMA,
            pltpu.SemaphoreType.DMA,
        ],
    )
    dummy = jnp.zeros((CH, WACC), jnp.float32)
    return f(t_tab, idxpack, idxs, dummy)


def _stageC_body(slo_ref, shi_ref, cnt_ref, x_ref, w2a_ref, w2b_ref, b2_ref,
                 wih_ref, bih_ref, whh_ref, bhh_ref, g_ref, be_ref, o_ref):
    bf = jnp.bfloat16
    cnt = cnt_ref[...]
    agg_sum = (
        jnp.dot(slo_ref[0].astype(bf), w2a_ref[...],
                preferred_element_type=jnp.float32)
        + jnp.dot(shi_ref[0].astype(bf), w2b_ref[...],
                  preferred_element_type=jnp.float32)
        + cnt * b2_ref[...]
    )
    agg = agg_sum / (cnt + 1e-8)
    xb = x_ref[...]
    gi = jnp.dot(agg.astype(bf), wih_ref[...],
                 preferred_element_type=jnp.float32) + bih_ref[...]
    gh = jnp.dot(xb.astype(bf), whh_ref[...],
                 preferred_element_type=jnp.float32) + bhh_ref[...]
    r = jax.nn.sigmoid(gi[:, :D] + gh[:, :D])
    z = jax.nn.sigmoid(gi[:, D:2 * D] + gh[:, D:2 * D])
    nn_ = jnp.tanh(gi[:, 2 * D:] + r * gh[:, 2 * D:])
    xn = (1.0 - z) * nn_ + z * xb
    mean = jnp.mean(xn, axis=-1, keepdims=True)
    xc = xn - mean
    var = jnp.mean(xc * xc, axis=-1, keepdims=True)
    o_ref[...] = xc * jax.lax.rsqrt(var + 1e-5) * g_ref[...] + be_ref[...]


def _stageC(feat, cnt, x, w2a, w2b, b2, wiht, bih, whht, bhh, gamma, beta):
    blk = 400
    grid = (N // blk,)
    full = lambda shape: pl.BlockSpec(shape, lambda i: tuple(0 for _ in shape))
    return pl.pallas_call(
        _stageC_body,
        grid=grid,
        in_specs=[
            pl.BlockSpec((1, blk, H), lambda i: (0, i, 0)),
            pl.BlockSpec((1, blk, H), lambda i: (1, i, 0)),
            pl.BlockSpec((blk, 1), lambda i: (i, 0)),
            pl.BlockSpec((blk, D), lambda i: (i, 0)),
            full((H, D)),
            full((H, D)),
            full((1, D)),
            full((D, 3 * D)),
            full((1, 3 * D)),
            full((D, 3 * D)),
            full((1, 3 * D)),
            full((1, D)),
            full((1, D)),
        ],
        out_specs=pl.BlockSpec((blk, D), lambda i: (i, 0)),
        out_shape=jax.ShapeDtypeStruct((N, D), jnp.float32),
    )(feat, feat, cnt, x, w2a, w2b, b2, wiht, bih, whht, bhh, gamma, beta)


def kernel(x, edge_index, W1, b1, W2, b2, W_ih, b_ih, W_hh, b_hh, gamma, beta):
    row = edge_index[0].astype(jnp.int32)
    col = edge_index[1].astype(jnp.int32)

    w1t = W1.T
    wstk = jnp.stack([
        w1t[:D, :H], w1t[:D, H:], w1t[D:, :H], w1t[D:, H:],
    ])
    zh = jnp.zeros((H,), jnp.float32)
    bstk = jnp.stack([b1[:H], b1[H:], zh, zh]).reshape(4, 1, H)
    t_tab = _stageA(x.astype(jnp.bfloat16), wstk.astype(jnp.bfloat16),
                    bstk)

    er = row.reshape(NS, NCHUNK, CH)
    ec = col.reshape(NS, NCHUNK, CH)
    pad = jnp.zeros((NS, NCHUNK, IOP - CH), jnp.int32)

    def _mk(cc):
        return jnp.concatenate(
            [er, pad, er + cc * N, pad, ec + (2 + cc) * N, pad], axis=-1)

    idxpack = jnp.stack([_mk(0), _mk(1)]).reshape(
        NC, NS, NCHUNK, 1, IXW)

    feat, cntp = _stageB(t_tab, idxpack,
                         row.reshape(NS, EPT // (5 * HCH), 5, HCH))
    cnt = cntp[0].reshape(-1)[:N].reshape(N, 1)

    bf = jnp.bfloat16
    w2t = W2.T
    return _stageC(
        feat, cnt, x,
        w2t[:H].astype(bf), w2t[H:].astype(bf), b2.reshape(1, D),
        W_ih.T.astype(bf), b_ih.reshape(1, 3 * D),
        W_hh.T.astype(bf), b_hh.reshape(1, 3 * D),
        gamma.reshape(1, D), beta.reshape(1, D),
    )


# speedup vs baseline: 4.3212x; 1.0235x over previous
import functools

import jax
import jax.numpy as jnp
from jax import lax
from jax.experimental import pallas as pl
from jax.experimental.pallas import tpu as pltpu
from jax.experimental.pallas import tpu_sc as plsc

N = 10000
E = 160000
D = 256
H = 128
WACC = 128
CROWS = 80
NACC = N + CROWS
NS = 16
NC = 2
EPT = E // NS
CH = 100
NCHUNK = EPT // CH
IOP = 128
IOQ = 256
IXW = 384
RPT = 624
TAIL = N - NS * RPT
ZB = 80
HCH = 80


def _stageA_body(x_ref, w_ref, b_ref, t_ref):
    t_ref[...] = (
        jnp.dot(x_ref[...], w_ref[0], preferred_element_type=jnp.float32)
        + b_ref[0]
    )


def _stageA(x, wstk, bstk):
    blk = 400
    grid = (4, N // blk)
    return pl.pallas_call(
        _stageA_body,
        grid=grid,
        in_specs=[
            pl.BlockSpec((blk, D), lambda i, j: (j, 0)),
            pl.BlockSpec((1, D, H), lambda i, j: (i, 0, 0)),
            pl.BlockSpec((1, 1, H), lambda i, j: (i, 0, 0)),
        ],
        out_specs=pl.BlockSpec((blk, H), lambda i, j: (i * (N // blk) + j, 0)),
        out_shape=jax.ShapeDtypeStruct((4 * N, H), jnp.float32),
    )(x, wstk, bstk)


def _stageB_tile(t_hbm, idxpack_hbm, idxs_hbm, dummy_hbm, feat_hbm, cnt_hbm,
                 ix0, ix1, ic_v, p0, p1, q_v, h_v,
                 shared, semp0, semp1, semq, semsc0, semsc1, semix, semh):
    c = lax.axis_index("c")
    s = lax.axis_index("s")
    ixs = (ix0, ix1)
    ps = (p0, p1)
    semps = (semp0, semp1)
    semscs = (semsc0, semsc1)

    zv = jnp.zeros((16,), jnp.float32)

    def zero_p0(k, carry):
        for l in range(WACC // 16):
            p0[k, pl.ds(l * 16, 16)] = zv
        return carry

    lax.fori_loop(0, CH, zero_p0, 0)

    for k in range(CROWS // 16):
        ic_v[0, pl.ds(k * 16, 16)] = lax.iota(jnp.int32, 16) + (N + k * 16)

    base = s * RPT

    def zero_acc(k, carry):
        pltpu.sync_copy(p0.at[pl.ds(0, ZB)],
                        shared.at[pl.ds(base + k * ZB, ZB)])
        return carry

    lax.fori_loop(0, 7, zero_acc, 0)
    pltpu.sync_copy(p0.at[pl.ds(0, 64)],
                    shared.at[pl.ds(base + 7 * ZB, 64)])

    @pl.when(s == NS - 1)
    def _():
        pltpu.sync_copy(p0.at[pl.ds(0, TAIL)],
                        shared.at[pl.ds(NS * RPT, TAIL)])

    @pl.when(s == 0)
    def _():
        pltpu.sync_copy(p0.at[pl.ds(0, CROWS)], shared.at[pl.ds(N, CROWS)])

    plsc.subcore_barrier()

    def compute_inplace(pb, qb):
        def row4(k4, rc):
            for r in range(4):
                k = 4 * k4 + r
                for l in range(H // 16):
                    p = pb[k, pl.ds(l * 16, 16)]
                    q = qb[k, pl.ds(l * 16, 16)]
                    h = p + q
                    pb[k, pl.ds(l * 16, 16)] = h / (1.0 + jnp.exp(-h))
            return rc

        lax.fori_loop(0, CH // 4, row4, 0)

    def wait_scatter(b):
        pltpu.make_async_copy(dummy_hbm, ps[b], semscs[b]).wait()

    def ix_sct(b):
        return ixs[b].at[0, pl.ds(0, CH)]

    def ix_p(b):
        return ixs[b].at[0, pl.ds(IOP, CH)]

    def ix_q(b):
        return ixs[b].at[0, pl.ds(IOQ, CH)]

    pltpu.sync_copy(idxpack_hbm.at[c, s, 0], ix0)
    pltpu.async_copy(t_hbm.at[ix_p(0)], p0, semp0)
    pltpu.async_copy(t_hbm.at[ix_q(0)], q_v, semq)

    def two_chunks(j2, carry):
        for b in (0, 1):
            j = 2 * j2 + b
            nb = 1 - b
            if b == 0:
                @pl.when(j2 > 0)
                def _():
                    wait_scatter(nb)
            else:
                wait_scatter(nb)
            pltpu.sync_copy(idxpack_hbm.at[c, s, j + 1], ixs[nb])
            pltpu.async_copy(t_hbm.at[ix_p(nb)], ps[nb], semps[nb])
            pltpu.make_async_copy(dummy_hbm, ps[b], semps[b]).wait()
            pltpu.make_async_copy(dummy_hbm, q_v, semq).wait()
            compute_inplace(ps[b], q_v)
            pltpu.async_copy(t_hbm.at[ix_q(nb)], q_v, semq)
            pltpu.async_copy(ps[b], shared.at[ix_sct(b)], semscs[b],
                             add=True)
        return carry

    lax.fori_loop(0, (NCHUNK - 2) // 2, two_chunks, 0)

    pltpu.make_async_copy(dummy_hbm, p0, semp0).wait()
    pltpu.make_async_copy(dummy_hbm, q_v, semq).wait()
    compute_inplace(p0, q_v)
    wait_scatter(1)
    pltpu.sync_copy(idxpack_hbm.at[c, s, NCHUNK - 1], ix1)
    pltpu.async_copy(t_hbm.at[ix_p(1)], p1, semp1)
    pltpu.async_copy(t_hbm.at[ix_q(1)], q_v, semq)
    pltpu.async_copy(p0, shared.at[ix_sct(0)], semsc0, add=True)
    pltpu.make_async_copy(dummy_hbm, p1, semp1).wait()
    pltpu.make_async_copy(dummy_hbm, q_v, semq).wait()
    compute_inplace(p1, q_v)
    pltpu.sync_copy(p1, shared.at[ix_sct(1)], add=True)
    wait_scatter(0)

    def zero_q(k, carry):
        for l in range(WACC // 16):
            q_v[k, pl.ds(l * 16, 16)] = zv
        return carry

    lax.fori_loop(0, CROWS, zero_q, 0)
    ones = jnp.ones((16,), jnp.float32)

    def hist(g, carry):
        pltpu.async_copy(idxs_hbm.at[s, g], h_v, semh).wait()
        def hrow(r, rc):
            for k in range(HCH // 16):
                iv = h_v[r, pl.ds(k * 16, 16)]
                plsc.addupdate_scatter(
                    q_v, [lax.shift_right_logical(iv, 7), iv & 127], ones)
            return rc
        lax.fori_loop(0, 5, hrow, 0)
        return carry

    lax.fori_loop(0, EPT // (5 * HCH), hist, 0)
    pltpu.sync_copy(q_v.at[pl.ds(0, CROWS)], shared.at[ic_v.at[0]],
                    add=True)

    plsc.subcore_barrier()

    pltpu.sync_copy(shared.at[pl.ds(base, RPT)],
                    feat_hbm.at[c, pl.ds(base, RPT)])

    @pl.when(s == NS - 1)
    def _():
        pltpu.sync_copy(shared.at[pl.ds(NS * RPT, TAIL)],
                        feat_hbm.at[c, pl.ds(NS * RPT, TAIL)])
        pltpu.sync_copy(shared.at[pl.ds(N, CROWS)], cnt_hbm.at[c])


def _stageB(t_tab, idxpack, idxs):
    mesh = plsc.VectorSubcoreMesh(core_axis_name="c", subcore_axis_name="s")
    f = pl.kernel(
        _stageB_tile,
        out_type=(jax.ShapeDtypeStruct((NC, N, WACC), jnp.float32),
                  jax.ShapeDtypeStruct((NC, CROWS, WACC), jnp.float32)),
        mesh=mesh,
        compiler_params=pltpu.CompilerParams(needs_layout_passes=False),
        scratch_types=[
            pltpu.VMEM((1, IXW), jnp.int32),
            pltpu.VMEM((1, IXW), jnp.int32),
            pltpu.VMEM((1, CROWS), jnp.int32),
            pltpu.VMEM((CH, H), jnp.float32),
            pltpu.VMEM((CH, H), jnp.float32),
            pltpu.VMEM((CH, WACC), jnp.float32),
            pltpu.VMEM((5, HCH), jnp.int32),
            pltpu.VMEM_SHARED((NACC, WACC), jnp.float32),
            pltpu.SemaphoreType.DMA,
            pltpu.SemaphoreType.DMA,
            pltpu.SemaphoreType.DMA,
            pltpu.SemaphoreType.DMA,
            pltpu.SemaphoreType.DMA,
            pltpu.SemaphoreType.DMA,
            pltpu.SemaphoreType.DMA,
        ],
    )
    dummy = jnp.zeros((CH, WACC), jnp.float32)
    return f(t_tab, idxpack, idxs, dummy)


def _stageC0_body(x_ref, whh_ref, bhh_ref, gh_ref):
    gh_ref[...] = jnp.dot(
        x_ref[...].astype(jnp.bfloat16), whh_ref[...],
        preferred_element_type=jnp.float32) + bhh_ref[...]


def _stageC0(x, whht, bhh):
    blk = 400
    full = lambda shape: pl.BlockSpec(shape, lambda i: tuple(0 for _ in shape))
    return pl.pallas_call(
        _stageC0_body,
        grid=(N // blk,),
        in_specs=[
            pl.BlockSpec((blk, D), lambda i: (i, 0)),
            full((D, 3 * D)),
            full((1, 3 * D)),
        ],
        out_specs=pl.BlockSpec((blk, 3 * D), lambda i: (i, 0)),
        out_shape=jax.ShapeDtypeStruct((N, 3 * D), jnp.float32),
    )(x, whht, bhh)


def _stageC_body(slo_ref, shi_ref, cnt_ref, x_ref, gh_ref, w2a_ref, w2b_ref,
                 b2_ref, wih_ref, bih_ref, g_ref, be_ref, o_ref):
    bf = jnp.bfloat16
    cnt = cnt_ref[...]
    agg_sum = (
        jnp.dot(slo_ref[0].astype(bf), w2a_ref[...],
                preferred_element_type=jnp.float32)
        + jnp.dot(shi_ref[0].astype(bf), w2b_ref[...],
                  preferred_element_type=jnp.float32)
        + cnt * b2_ref[...]
    )
    agg = agg_sum / (cnt + 1e-8)
    xb = x_ref[...]
    gi = jnp.dot(agg.astype(bf), wih_ref[...],
                 preferred_element_type=jnp.float32) + bih_ref[...]
    gh = gh_ref[...]
    r = jax.nn.sigmoid(gi[:, :D] + gh[:, :D])
    z = jax.nn.sigmoid(gi[:, D:2 * D] + gh[:, D:2 * D])
    nn_ = jnp.tanh(gi[:, 2 * D:] + r * gh[:, 2 * D:])
    xn = (1.0 - z) * nn_ + z * xb
    mean = jnp.mean(xn, axis=-1, keepdims=True)
    xc = xn - mean
    var = jnp.mean(xc * xc, axis=-1, keepdims=True)
    o_ref[...] = xc * jax.lax.rsqrt(var + 1e-5) * g_ref[...] + be_ref[...]


def _stageC(feat, cnt, x, gh, w2a, w2b, b2, wiht, bih, gamma, beta):
    blk = 400
    grid = (N // blk,)
    full = lambda shape: pl.BlockSpec(shape, lambda i: tuple(0 for _ in shape))
    return pl.pallas_call(
        _stageC_body,
        grid=grid,
        in_specs=[
            pl.BlockSpec((1, blk, H), lambda i: (0, i, 0)),
            pl.BlockSpec((1, blk, H), lambda i: (1, i, 0)),
            pl.BlockSpec((blk, 1), lambda i: (i, 0)),
            pl.BlockSpec((blk, D), lambda i: (i, 0)),
            pl.BlockSpec((blk, 3 * D), lambda i: (i, 0)),
            full((H, D)),
            full((H, D)),
            full((1, D)),
            full((D, 3 * D)),
            full((1, 3 * D)),
            full((1, D)),
            full((1, D)),
        ],
        out_specs=pl.BlockSpec((blk, D), lambda i: (i, 0)),
        out_shape=jax.ShapeDtypeStruct((N, D), jnp.float32),
    )(feat, feat, cnt, x, gh, w2a, w2b, b2, wiht, bih, gamma, beta)


def kernel(x, edge_index, W1, b1, W2, b2, W_ih, b_ih, W_hh, b_hh, gamma, beta):
    row = edge_index[0].astype(jnp.int32)
    col = edge_index[1].astype(jnp.int32)

    w1t = W1.T
    wstk = jnp.stack([
        w1t[:D, :H], w1t[:D, H:], w1t[D:, :H], w1t[D:, H:],
    ])
    zh = jnp.zeros((H,), jnp.float32)
    bstk = jnp.stack([b1[:H], b1[H:], zh, zh]).reshape(4, 1, H)
    t_tab = _stageA(x.astype(jnp.bfloat16), wstk.astype(jnp.bfloat16),
                    bstk)

    er = row.reshape(NS, NCHUNK, CH)
    ec = col.reshape(NS, NCHUNK, CH)
    pad = jnp.zeros((NS, NCHUNK, IOP - CH), jnp.int32)

    def _mk(cc):
        return jnp.concatenate(
            [er, pad, er + cc * N, pad, ec + (2 + cc) * N, pad], axis=-1)

    idxpack = jnp.stack([_mk(0), _mk(1)]).reshape(
        NC, NS, NCHUNK, 1, IXW)

    feat, cntp = _stageB(t_tab, idxpack,
                         row.reshape(NS, EPT // (5 * HCH), 5, HCH))
    cnt = cntp[0].reshape(-1)[:N].reshape(N, 1)

    bf = jnp.bfloat16
    w2t = W2.T
    gh = _stageC0(x, W_hh.T.astype(bf), b_hh.reshape(1, 3 * D))
    return _stageC(
        feat, cnt, x, gh,
        w2t[:H].astype(bf), w2t[H:].astype(bf), b2.reshape(1, D),
        W_ih.T.astype(bf), b_ih.reshape(1, 3 * D),
        gamma.reshape(1, D), beta.reshape(1, D),
    )
